# baseline (speedup 1.0000x reference)
import functools

import jax
import jax.numpy as jnp
from jax import lax
from jax.experimental import pallas as pl
from jax.experimental.pallas import tpu as pltpu
from jax.experimental.pallas import tpu_sc as plsc

VOCAB = 1024
EMBED = 64
N_ROWS = 16 * 32 * 32
N_HALF = N_ROWS // 2
ROWS_PER_BLOCK = 2048
BLOCKS_PER_HALF = N_HALF // ROWS_PER_BLOCK
IMGS_PER_BLOCK = ROWS_PER_BLOCK // VOCAB

NUM_SC_CORES = 2
NUM_SUBCORES = 16
NUM_WORKERS = NUM_SC_CORES * NUM_SUBCORES
ROWS_PER_WORKER = N_HALF // NUM_WORKERS


def _argmin_body_cb(z_ref, cb_ref, idx_ref, cbp_ref):
    z = z_ref[...]
    cb = cb_ref[...]
    z_sq = jnp.sum(z * z, axis=1, keepdims=True)
    e_sq = jnp.sum(cb * cb, axis=1, keepdims=True)
    scores2 = lax.dot_general(cb, z + z, (((1,), (1,)), ((), ())))
    d = e_sq + z_sq.T - scores2
    first = jnp.argmin(d, axis=0)
    idx_ref[0, :, :] = first.astype(jnp.int32).reshape(IMGS_PER_BLOCK, VOCAB)

    @pl.when(pl.program_id(0) == 0)
    def _():
        cbp_ref[...] = jnp.concatenate(
            [cb, jnp.zeros((VOCAB, 128 - EMBED), jnp.float32)], axis=1)


def _argmin_body(z_ref, cb_ref, idx_ref):
    z = z_ref[...]
    cb = cb_ref[...]
    z_sq = jnp.sum(z * z, axis=1, keepdims=True)
    e_sq = jnp.sum(cb * cb, axis=1, keepdims=True)
    scores2 = lax.dot_general(cb, z + z, (((1,), (1,)), ((), ())))
    d = e_sq + z_sq.T - scores2
    first = jnp.argmin(d, axis=0)
    idx_ref[0, :, :] = first.astype(jnp.int32).reshape(IMGS_PER_BLOCK, VOCAB)


_IDX_SHAPE = jax.ShapeDtypeStruct(
    (BLOCKS_PER_HALF, IMGS_PER_BLOCK, VOCAB), jnp.int32)
_IN_SPECS = [
    pl.BlockSpec((ROWS_PER_BLOCK, EMBED), lambda i: (i, 0)),
    pl.BlockSpec((VOCAB, EMBED), lambda i: (0, 0)),
]
_IDX_SPEC = pl.BlockSpec((1, IMGS_PER_BLOCK, VOCAB), lambda i: (i, 0, 0))


def _argmin_half_cb(z_half, codebook):
    return pl.pallas_call(
        _argmin_body_cb,
        grid=(BLOCKS_PER_HALF,),
        in_specs=_IN_SPECS,
        out_specs=[_IDX_SPEC, pl.BlockSpec((VOCAB, 128), lambda i: (0, 0))],
        out_shape=[_IDX_SHAPE,
                   jax.ShapeDtypeStruct((VOCAB, 128), jnp.float32)],
    )(z_half, codebook)


def _argmin_half(z_half, codebook):
    return pl.pallas_call(
        _argmin_body,
        grid=(BLOCKS_PER_HALF,),
        in_specs=_IN_SPECS,
        out_specs=_IDX_SPEC,
        out_shape=_IDX_SHAPE,
    )(z_half, codebook)


@functools.cache
def _make_sc_gather():
    mesh = plsc.VectorSubcoreMesh(core_axis_name="c", subcore_axis_name="s")

    @functools.partial(
        pl.kernel,
        mesh=mesh,
        out_type=jax.ShapeDtypeStruct((N_HALF, 128), jnp.float32),
        scratch_types=[
            pltpu.VMEM((ROWS_PER_WORKER,), jnp.int32),
            pltpu.VMEM((ROWS_PER_WORKER, 128), jnp.float32),
            pltpu.SemaphoreType.DMA,
        ],
    )
    def _sc_gather(cbp_hbm, idx_hbm, out_hbm, idx_v, rows_v, sem):
        wid = lax.axis_index("s") * NUM_SC_CORES + lax.axis_index("c")
        base = wid * ROWS_PER_WORKER
        blk = base // ROWS_PER_BLOCK
        rem = base % ROWS_PER_BLOCK
        img = rem // VOCAB
        off = rem % VOCAB
        pltpu.sync_copy(idx_hbm.at[blk, img, pl.ds(off, ROWS_PER_WORKER)],
                        idx_v)
        pltpu.async_copy(cbp_hbm.at[idx_v], rows_v, sem).wait()
        pltpu.sync_copy(rows_v, out_hbm.at[pl.ds(base, ROWS_PER_WORKER)])

    return _sc_gather


def kernel(z, codebook):
    B, H, W, D = z.shape
    z_flat = z.reshape(-1, D)
    gather = _make_sc_gather()

    idx0, cb_pad = _argmin_half_cb(z_flat[:N_HALF], codebook)
    zq0 = gather(cb_pad, idx0)
    idx1 = _argmin_half(z_flat[N_HALF:], codebook)
    zq1 = gather(cb_pad, idx1)

    indices = jnp.concatenate(
        [idx0.reshape(B // 2, H, W), idx1.reshape(B // 2, H, W)], axis=0)
    z_q = jnp.concatenate(
        [zq0[:, :D].reshape(B // 2, H, W, D),
         zq1[:, :D].reshape(B // 2, H, W, D)], axis=0)
    return (z_q, indices)


# speedup vs baseline: 1.4830x; 1.4830x over previous
import functools

import jax
import jax.numpy as jnp
from jax import lax
from jax.experimental import pallas as pl
from jax.experimental.pallas import tpu as pltpu
from jax.experimental.pallas import tpu_sc as plsc

VOCAB = 1024
EMBED = 64
N_ROWS = 16 * 32 * 32
ROWS_PER_BLOCK = 2048
NUM_BLOCKS = N_ROWS // ROWS_PER_BLOCK
IMGS_PER_BLOCK = ROWS_PER_BLOCK // VOCAB

NUM_SC_CORES = 2
NUM_SUBCORES = 16
NUM_WORKERS = NUM_SC_CORES * NUM_SUBCORES
ROWS_PER_WORKER = N_ROWS // NUM_WORKERS


def _argmin_body(z_ref, cbt_ref, idx_ref, cbp_ref, cb_vmem, esq_vmem):
    @pl.when(pl.program_id(0) == 0)
    def _():
        cb0 = cbt_ref[...].T
        cb_vmem[...] = cb0
        esq_vmem[...] = jnp.sum(cb0 * cb0, axis=1, keepdims=True)
        cbp_ref[...] = jnp.concatenate(
            [cb0, jnp.zeros((VOCAB, 128 - EMBED), jnp.float32)], axis=1)

    z = z_ref[...]
    cb = cb_vmem[...]
    z_sq = jnp.sum(z * z, axis=1, keepdims=True)
    e_sq = esq_vmem[...]
    scores2 = lax.dot_general(cb, z + z, (((1,), (1,)), ((), ())))
    d = e_sq + z_sq.T - scores2
    first = jnp.argmin(d, axis=0)
    idx_ref[0, :, :] = first.astype(jnp.int32).reshape(IMGS_PER_BLOCK, VOCAB)


def _argmin_indices(z_flat, codebook):
    idx, cb_pad = pl.pallas_call(
        _argmin_body,
        grid=(NUM_BLOCKS,),
        in_specs=[
            pl.BlockSpec((ROWS_PER_BLOCK, EMBED), lambda i: (i, 0)),
            pl.BlockSpec((EMBED, VOCAB), lambda i: (0, 0)),
        ],
        out_specs=[
            pl.BlockSpec((1, IMGS_PER_BLOCK, VOCAB), lambda i: (i, 0, 0)),
            pl.BlockSpec((VOCAB, 128), lambda i: (0, 0)),
        ],
        out_shape=[
            jax.ShapeDtypeStruct((NUM_BLOCKS, IMGS_PER_BLOCK, VOCAB),
                                 jnp.int32),
            jax.ShapeDtypeStruct((VOCAB, 128), jnp.float32),
        ],
        scratch_shapes=[
            pltpu.VMEM((VOCAB, EMBED), jnp.float32),
            pltpu.VMEM((VOCAB, 1), jnp.float32),
        ],
    )(z_flat, codebook.T)
    return idx, cb_pad


@functools.cache
def _make_sc_gather():
    mesh = plsc.VectorSubcoreMesh(core_axis_name="c", subcore_axis_name="s")

    @functools.partial(
        pl.kernel,
        mesh=mesh,
        out_type=jax.ShapeDtypeStruct((N_ROWS, 128), jnp.float32),
        scratch_types=[
            pltpu.VMEM((ROWS_PER_WORKER,), jnp.int32),
            pltpu.VMEM((ROWS_PER_WORKER, 128), jnp.float32),
            pltpu.SemaphoreType.DMA,
        ],
    )
    def _sc_gather(cbp_hbm, idx_hbm, out_hbm, idx_v, rows_v, sem):
        wid = lax.axis_index("s") * NUM_SC_CORES + lax.axis_index("c")
        base = wid * ROWS_PER_WORKER
        blk = base // ROWS_PER_BLOCK
        rem = base % ROWS_PER_BLOCK
        img = rem // VOCAB
        off = rem % VOCAB
        pltpu.sync_copy(idx_hbm.at[blk, img, pl.ds(off, ROWS_PER_WORKER)],
                        idx_v)
        pltpu.async_copy(cbp_hbm.at[idx_v], rows_v, sem).wait()
        pltpu.sync_copy(rows_v, out_hbm.at[pl.ds(base, ROWS_PER_WORKER)])

    return _sc_gather


def kernel(z, codebook):
    B, H, W, D = z.shape
    z_flat = z.reshape(-1, D)
    idx, cb_pad = _argmin_indices(z_flat, codebook)
    indices = idx.reshape(B, H, W)
    z_q = _make_sc_gather()(cb_pad, idx)[:, :D].reshape(B, H, W, D)
    return (z_q, indices)
